# baseline (speedup 1.0000x reference)
import jax
import jax.numpy as jnp
from jax.experimental import pallas as pl
from jax.experimental.pallas import tpu as pltpu

_ANCHOR_W = (10.0, 16.0, 33.0)
_ANCHOR_H = (13.0, 30.0, 23.0)
_A = 3
_NC = 85
_IMG_DIM = 416.0


def _decode_block(v, a_idx, grid_g, stride):
    gg = v.shape[1]
    sig = jax.nn.sigmoid(v)
    top = v[0:8]
    ee = jnp.exp(jnp.exp(top))

    lane = jax.lax.broadcasted_iota(jnp.int32, (8, gg), 1)
    col = (lane % grid_g).astype(jnp.float32)
    row = (lane // grid_g).astype(jnp.float32)

    aw = jnp.where(a_idx == 0, _ANCHOR_W[0],
                   jnp.where(a_idx == 1, _ANCHOR_W[1], _ANCHOR_W[2]))
    ah = jnp.where(a_idx == 0, _ANCHOR_H[0],
                   jnp.where(a_idx == 1, _ANCHOR_H[1], _ANCHOR_H[2]))

    ch = jax.lax.broadcasted_iota(jnp.int32, (8, gg), 0)
    sig8 = sig[0:8]
    res8 = jnp.where(
        ch == 0, (sig8 + col) * stride,
        jnp.where(ch == 1, (sig8 + row) * stride,
                  jnp.where(ch == 2, ee * aw,
                            jnp.where(ch == 3, ee * ah, sig8))))
    full = jnp.concatenate([res8, sig[8:]], axis=0)
    return full.T


def _yolo_kernel(in_ref, out_ref, *, grid_g, stride):
    a_idx = pl.program_id(1)
    out_ref[0] = _decode_block(in_ref[0, 0], a_idx, grid_g, stride)


def kernel(input_):
    B, C, G, _ = input_.shape
    gg = G * G
    stride = _IMG_DIM / G
    x = input_.reshape(B, _A, _NC, gg)

    import functools
    return pl.pallas_call(
        functools.partial(_yolo_kernel, grid_g=G, stride=stride),
        grid=(B, _A),
        in_specs=[pl.BlockSpec((1, 1, _NC, gg),
                               lambda b, a: (b, a, 0, 0))],
        out_specs=pl.BlockSpec((1, gg, _NC), lambda b, a: (b, a, 0)),
        out_shape=jax.ShapeDtypeStruct((B, _A * gg, _NC), jnp.float32),
        compiler_params=pltpu.CompilerParams(
            dimension_semantics=("parallel", "parallel")),
    )(x)


# speedup vs baseline: 1.4587x; 1.4587x over previous
import jax
import jax.numpy as jnp
from jax.experimental import pallas as pl
from jax.experimental.pallas import tpu as pltpu

_ANCHOR_W = (10.0, 16.0, 33.0)
_ANCHOR_H = (13.0, 30.0, 23.0)
_A = 3
_NC = 85
_IMG_DIM = 416.0


def _decode_block(v, a_idx, grid_g, stride):
    gg = v.shape[1]
    sig = jax.nn.sigmoid(v)
    top = v[0:8]
    ee = jnp.exp(jnp.exp(top))

    lane = jax.lax.broadcasted_iota(jnp.int32, (8, gg), 1)
    col = (lane % grid_g).astype(jnp.float32)
    row = (lane // grid_g).astype(jnp.float32)

    aw = jnp.where(a_idx == 0, _ANCHOR_W[0],
                   jnp.where(a_idx == 1, _ANCHOR_W[1], _ANCHOR_W[2]))
    ah = jnp.where(a_idx == 0, _ANCHOR_H[0],
                   jnp.where(a_idx == 1, _ANCHOR_H[1], _ANCHOR_H[2]))

    ch = jax.lax.broadcasted_iota(jnp.int32, (8, gg), 0)
    sig8 = sig[0:8]
    res8 = jnp.where(
        ch == 0, (sig8 + col) * stride,
        jnp.where(ch == 1, (sig8 + row) * stride,
                  jnp.where(ch == 2, ee * aw,
                            jnp.where(ch == 3, ee * ah, sig8))))
    full = jnp.concatenate([res8, sig[8:]], axis=0)
    return full.T


def _yolo_kernel(in_ref, out_ref, *, grid_g, stride):
    a_idx = pl.program_id(1)
    v = in_ref[0].reshape(_NC, grid_g * grid_g)
    out_ref[0] = _decode_block(v, a_idx, grid_g, stride)


def kernel(input_):
    B, C, G, _ = input_.shape
    gg = G * G
    stride = _IMG_DIM / G

    import functools
    return pl.pallas_call(
        functools.partial(_yolo_kernel, grid_g=G, stride=stride),
        grid=(B, _A),
        in_specs=[pl.BlockSpec((1, _NC, G, G),
                               lambda b, a: (b, a, 0, 0))],
        out_specs=pl.BlockSpec((1, gg, _NC), lambda b, a: (b, a, 0)),
        out_shape=jax.ShapeDtypeStruct((B, _A * gg, _NC), jnp.float32),
        compiler_params=pltpu.CompilerParams(
            dimension_semantics=("parallel", "parallel")),
    )(input_)


# speedup vs baseline: 1.5756x; 1.0801x over previous
import jax
import jax.numpy as jnp
from jax.experimental import pallas as pl
from jax.experimental.pallas import tpu as pltpu

_ANCHOR_W = (10.0, 16.0, 33.0)
_ANCHOR_H = (13.0, 30.0, 23.0)
_A = 3
_NC = 85
_IMG_DIM = 416.0


def _decode_block(v, a_idx, grid_g, stride):
    gg = v.shape[1]
    sig = jax.nn.sigmoid(v)
    top = v[0:8]
    ee = jnp.exp(jnp.exp(top))

    lane = jax.lax.broadcasted_iota(jnp.int32, (8, gg), 1)
    col = (lane % grid_g).astype(jnp.float32)
    row = (lane // grid_g).astype(jnp.float32)

    aw = jnp.where(a_idx == 0, _ANCHOR_W[0],
                   jnp.where(a_idx == 1, _ANCHOR_W[1], _ANCHOR_W[2]))
    ah = jnp.where(a_idx == 0, _ANCHOR_H[0],
                   jnp.where(a_idx == 1, _ANCHOR_H[1], _ANCHOR_H[2]))

    ch = jax.lax.broadcasted_iota(jnp.int32, (8, gg), 0)
    sig8 = sig[0:8]
    res8 = jnp.where(
        ch == 0, (sig8 + col) * stride,
        jnp.where(ch == 1, (sig8 + row) * stride,
                  jnp.where(ch == 2, ee * aw,
                            jnp.where(ch == 3, ee * ah, sig8))))
    full = jnp.concatenate([res8, sig[8:]], axis=0)
    return full.T


_NB = 2


def _yolo_kernel(in_ref, out_ref, *, grid_g, stride):
    a_idx = pl.program_id(1)
    for nb in range(_NB):
        v = in_ref[nb].reshape(_NC, grid_g * grid_g)
        out_ref[nb] = _decode_block(v, a_idx, grid_g, stride)


def kernel(input_):
    B, C, G, _ = input_.shape
    gg = G * G
    stride = _IMG_DIM / G

    import functools
    return pl.pallas_call(
        functools.partial(_yolo_kernel, grid_g=G, stride=stride),
        grid=(B // _NB, _A),
        in_specs=[pl.BlockSpec((_NB, _NC, G, G),
                               lambda b, a: (b, a, 0, 0))],
        out_specs=pl.BlockSpec((_NB, gg, _NC), lambda b, a: (b, a, 0)),
        out_shape=jax.ShapeDtypeStruct((B, _A * gg, _NC), jnp.float32),
        compiler_params=pltpu.CompilerParams(
            dimension_semantics=("parallel", "parallel")),
    )(input_)


# speedup vs baseline: 1.6378x; 1.0394x over previous
import jax
import jax.numpy as jnp
from jax.experimental import pallas as pl
from jax.experimental.pallas import tpu as pltpu

_ANCHOR_W = (10.0, 16.0, 33.0)
_ANCHOR_H = (13.0, 30.0, 23.0)
_A = 3
_NC = 85
_IMG_DIM = 416.0


def _decode_block(v, a_idx, grid_g, stride):
    gg = v.shape[1]
    sig = jax.nn.sigmoid(v)
    top = v[0:8]
    ee = jnp.exp(jnp.exp(top))

    lane = jax.lax.broadcasted_iota(jnp.int32, (8, gg), 1)
    col = (lane % grid_g).astype(jnp.float32)
    row = (lane // grid_g).astype(jnp.float32)

    aw = jnp.where(a_idx == 0, _ANCHOR_W[0],
                   jnp.where(a_idx == 1, _ANCHOR_W[1], _ANCHOR_W[2]))
    ah = jnp.where(a_idx == 0, _ANCHOR_H[0],
                   jnp.where(a_idx == 1, _ANCHOR_H[1], _ANCHOR_H[2]))

    ch = jax.lax.broadcasted_iota(jnp.int32, (8, gg), 0)
    sig8 = sig[0:8]
    res8 = jnp.where(
        ch == 0, (sig8 + col) * stride,
        jnp.where(ch == 1, (sig8 + row) * stride,
                  jnp.where(ch == 2, ee * aw,
                            jnp.where(ch == 3, ee * ah, sig8))))
    full = jnp.concatenate([res8, sig[8:]], axis=0)
    return full.T


_NB = 4


def _yolo_kernel(in_ref, out_ref, *, grid_g, stride):
    a_idx = pl.program_id(1)
    for nb in range(_NB):
        v = in_ref[nb].reshape(_NC, grid_g * grid_g)
        out_ref[nb] = _decode_block(v, a_idx, grid_g, stride)


def kernel(input_):
    B, C, G, _ = input_.shape
    gg = G * G
    stride = _IMG_DIM / G

    import functools
    return pl.pallas_call(
        functools.partial(_yolo_kernel, grid_g=G, stride=stride),
        grid=(B // _NB, _A),
        in_specs=[pl.BlockSpec((_NB, _NC, G, G),
                               lambda b, a: (b, a, 0, 0))],
        out_specs=pl.BlockSpec((_NB, gg, _NC), lambda b, a: (b, a, 0)),
        out_shape=jax.ShapeDtypeStruct((B, _A * gg, _NC), jnp.float32),
        compiler_params=pltpu.CompilerParams(
            dimension_semantics=("parallel", "parallel")),
    )(input_)
